# baseline (speedup 1.0000x reference)
import functools

import jax
import jax.numpy as jnp
from jax import lax
from jax.experimental import pallas as pl
from jax.experimental.pallas import tpu as pltpu

B, N, D_STATE, D_IN, M = 128, 8192, 25, 16, 128
BN = B * N
RBLK = 8192


def _predict_body(w_ref, b_ref, shift_ref, x_ref, out_ref):
    x = x_ref[...]
    xb = x.astype(jnp.bfloat16)
    w = w_ref[...].astype(jnp.bfloat16)
    logits = lax.dot_general(
        w, xb,
        dimension_numbers=(((1,), (1,)), ((), ())),
        preferred_element_type=jnp.float32,
    )
    logits = logits + b_ref[0, 0]
    p = 1.0 / (1.0 + jnp.exp(-logits))
    masked = jnp.where(p >= 0.5, p, 0.0) + shift_ref[0, 0]
    out_ref[...] = masked


def _predicts(states2d, W, b, shift):
    grid = BN // RBLK
    return pl.pallas_call(
        _predict_body,
        grid=(grid,),
        in_specs=[
            pl.BlockSpec((1, D_STATE), lambda i: (0, 0)),
            pl.BlockSpec((1, 1), lambda i: (0, 0)),
            pl.BlockSpec((1, 1), lambda i: (0, 0)),
            pl.BlockSpec((RBLK, D_STATE), lambda i: (i, 0)),
        ],
        out_specs=pl.BlockSpec((1, RBLK), lambda i: (0, i)),
        out_shape=jax.ShapeDtypeStruct((1, BN), jnp.float32),
    )(W.reshape(1, D_STATE), b.reshape(1, 1), shift.reshape(1, 1), states2d)


def kernel(states, inputs, W, b, k):
    shift = (jnp.asarray(k) - M).astype(jnp.float32)
    P = _predicts(states.reshape(BN, D_STATE), W, b, shift).reshape(B, N)
    topk_vals, topk_idx = lax.top_k(P, M)
    selected = jnp.take_along_axis(inputs, topk_idx[:, :, None], axis=1)
    log_actions = jnp.log(jnp.clip(topk_vals, 1e-8, 1.0))
    return selected * log_actions[:, :, None]


# speedup vs baseline: 1.1740x; 1.1740x over previous
import functools

import jax
import jax.numpy as jnp
from jax import lax
from jax.experimental import pallas as pl
from jax.experimental.pallas import tpu as pltpu

B, N, D_STATE, D_IN, M = 128, 8192, 25, 16, 128
BN = B * N
CBLK = 32768
RB = 8192


def _logits_body(w_ref, x_ref, out_ref):
    xb = x_ref[...].astype(jnp.bfloat16)
    wb = w_ref[...].astype(jnp.bfloat16)
    out_ref[...] = lax.dot_general(
        wb, xb,
        dimension_numbers=(((1,), (0,)), ((), ())),
        preferred_element_type=jnp.float32,
    )


def _mask_body(b_ref, shift_ref, x_ref, out_ref):
    logits = x_ref[...] + b_ref[0, 0]
    p = 1.0 / (1.0 + jnp.exp(-logits))
    out_ref[...] = jnp.where(p >= 0.5, p, 0.0) + shift_ref[0, 0]


def _predicts(states2d, W, b, shift):
    xT = states2d.T
    logits = pl.pallas_call(
        _logits_body,
        grid=(BN // CBLK,),
        in_specs=[
            pl.BlockSpec((1, D_STATE), lambda i: (0, 0)),
            pl.BlockSpec((D_STATE, CBLK), lambda i: (0, i)),
        ],
        out_specs=pl.BlockSpec((1, CBLK), lambda i: (0, i)),
        out_shape=jax.ShapeDtypeStruct((1, BN), jnp.float32),
    )(W.reshape(1, D_STATE), xT)
    return pl.pallas_call(
        _mask_body,
        grid=(BN // 128 // RB,),
        in_specs=[
            pl.BlockSpec((1, 1), lambda i: (0, 0)),
            pl.BlockSpec((1, 1), lambda i: (0, 0)),
            pl.BlockSpec((RB, 128), lambda i: (i, 0)),
        ],
        out_specs=pl.BlockSpec((RB, 128), lambda i: (i, 0)),
        out_shape=jax.ShapeDtypeStruct((BN // 128, 128), jnp.float32),
    )(b.reshape(1, 1), shift.reshape(1, 1), logits.reshape(BN // 128, 128))


def kernel(states, inputs, W, b, k):
    shift = (jnp.asarray(k) - M).astype(jnp.float32)
    P = _predicts(states.reshape(BN, D_STATE), W, b, shift).reshape(B, N)
    topk_vals, topk_idx = lax.top_k(P, M)
    selected = jnp.take_along_axis(inputs, topk_idx[:, :, None], axis=1)
    log_actions = jnp.log(jnp.clip(topk_vals, 1e-8, 1.0))
    return selected * log_actions[:, :, None]
